# baseline (speedup 1.0000x reference)
import jax
import jax.numpy as jnp
from jax.experimental import pallas as pl
from jax.experimental.pallas import tpu as pltpu


def _body(se_ref, sp_ref, x_ref, dw_ref, db_ref, uw_ref, o_ref):
    xb = x_ref[0]
    z = jnp.dot(xb, dw_ref[0, 0], preferred_element_type=jnp.float32)
    z = z + db_ref[0, 0, 0][None, :]
    z = z * jax.nn.sigmoid(z)
    o_ref[0, 0] = jnp.dot(z, uw_ref[0, 0], preferred_element_type=jnp.float32)


def kernel(x, expert_index, down_w, down_b, up_w):
    B, S, C = x.shape
    M, N, _, D = down_w.shape

    order = jnp.argsort(expert_index, axis=1).astype(jnp.int32)
    se = jnp.take_along_axis(expert_index, order, axis=1).astype(jnp.int32)

    db4 = down_b.reshape(M, N, 1, D)

    grid_spec = pltpu.PrefetchScalarGridSpec(
        num_scalar_prefetch=2,
        grid=(M, B),
        in_specs=[
            pl.BlockSpec((1, S, C), lambda m, b, se, sp: (sp[m, b], 0, 0)),
            pl.BlockSpec((1, 1, C, D), lambda m, b, se, sp: (m, se[m, b], 0, 0)),
            pl.BlockSpec((1, 1, 1, D), lambda m, b, se, sp: (m, se[m, b], 0, 0)),
            pl.BlockSpec((1, 1, D, C), lambda m, b, se, sp: (m, se[m, b], 0, 0)),
        ],
        out_specs=pl.BlockSpec((1, 1, S, C), lambda m, b, se, sp: (m, sp[m, b], 0, 0)),
    )

    out = pl.pallas_call(
        _body,
        grid_spec=grid_spec,
        out_shape=jax.ShapeDtypeStruct((M, B, S, C), jnp.float32),
        compiler_params=pltpu.CompilerParams(
            dimension_semantics=("arbitrary", "arbitrary"),
        ),
    )(se, order, x, down_w, db4, up_w)
    return out


# speedup vs baseline: 1.1300x; 1.1300x over previous
import jax
import jax.numpy as jnp
from jax.experimental import pallas as pl
from jax.experimental.pallas import tpu as pltpu


def _body(se_ref, sp_ref, x_ref, dw_ref, db_ref, uw_ref, o_ref):
    m = pl.program_id(0)
    b = pl.program_id(1)
    row = sp_ref[m, b]
    xb = x_ref[row]
    z = jnp.dot(xb, dw_ref[0, 0], preferred_element_type=jnp.float32)
    z = z + db_ref[0, 0, 0][None, :]
    z = z * jax.nn.sigmoid(z)
    o_ref[0, row] = jnp.dot(z, uw_ref[0, 0], preferred_element_type=jnp.float32)


def kernel(x, expert_index, down_w, down_b, up_w):
    B, S, C = x.shape
    M, N, _, D = down_w.shape

    order = jnp.argsort(expert_index, axis=1).astype(jnp.int32)
    se = jnp.take_along_axis(expert_index, order, axis=1).astype(jnp.int32)

    db4 = down_b.reshape(M, N, 1, D)

    grid_spec = pltpu.PrefetchScalarGridSpec(
        num_scalar_prefetch=2,
        grid=(M, B),
        in_specs=[
            pl.BlockSpec((B, S, C), lambda m, b, se, sp: (0, 0, 0)),
            pl.BlockSpec((1, 1, C, D), lambda m, b, se, sp: (m, se[m, b], 0, 0)),
            pl.BlockSpec((1, 1, 1, D), lambda m, b, se, sp: (m, se[m, b], 0, 0)),
            pl.BlockSpec((1, 1, D, C), lambda m, b, se, sp: (m, se[m, b], 0, 0)),
        ],
        out_specs=pl.BlockSpec((1, B, S, C), lambda m, b, se, sp: (m, 0, 0, 0)),
    )

    out = pl.pallas_call(
        _body,
        grid_spec=grid_spec,
        out_shape=jax.ShapeDtypeStruct((M, B, S, C), jnp.float32),
        compiler_params=pltpu.CompilerParams(
            dimension_semantics=("arbitrary", "arbitrary"),
        ),
    )(se, order, x, down_w, db4, up_w)
    return out


# speedup vs baseline: 1.3659x; 1.2088x over previous
import jax
import jax.numpy as jnp
from jax.experimental import pallas as pl
from jax.experimental.pallas import tpu as pltpu

_G = 8


def _body(ge_ref, rows_ref, qc_ref, x_ref, dw_ref, db_ref, uw_ref, o_ref):
    m = pl.program_id(0)
    g = pl.program_id(1)
    S = x_ref.shape[1]

    @pl.when(qc_ref[m, g] > 0)
    def _():
        rows = [rows_ref[m, g * _G + i] for i in range(_G)]
        xt = jnp.concatenate([x_ref[r] for r in rows], axis=0)
        z = jnp.dot(xt, dw_ref[0, 0], preferred_element_type=jnp.float32)
        z = z + db_ref[0, 0, 0][None, :]
        z = z * jax.nn.sigmoid(z)
        u = jnp.dot(z, uw_ref[0, 0], preferred_element_type=jnp.float32)
        for i in range(_G):
            o_ref[0, rows[i]] = u[i * S:(i + 1) * S]


def _routing(expert_index, N, NG):
    M, B = expert_index.shape
    order = jnp.argsort(expert_index, axis=1).astype(jnp.int32)
    se = jnp.take_along_axis(expert_index, order, axis=1).astype(jnp.int32)

    onehot = jax.nn.one_hot(expert_index, N, dtype=jnp.int32)
    counts = jnp.sum(onehot, axis=1)
    estart = jnp.cumsum(counts, axis=1) - counts
    gbefore = jnp.cumsum((counts + _G - 1) // _G, axis=1) \
        - (counts + _G - 1) // _G

    j = jnp.arange(B)[None, :]
    within = j - jnp.take_along_axis(estart, se, axis=1)
    gid = jnp.take_along_axis(gbefore, se, axis=1) + within // _G
    slot = within % _G

    rowid = jnp.arange(M)[:, None]
    ge = jnp.full((M, NG), -1, jnp.int32).at[rowid, gid].max(se)
    ge = jnp.where(ge < 0, se[:, -1:], ge)
    qc = jnp.zeros((M, NG), jnp.int32).at[rowid, gid].add(1)

    rows = jnp.zeros((M, NG * _G), jnp.int32).at[rowid, gid * _G + slot].set(order)
    gidx = jnp.arange(NG * _G)[None, :] // _G
    sidx = jnp.arange(NG * _G)[None, :] % _G
    qcg = jnp.take_along_axis(qc, gidx, axis=1)
    src = gidx * _G + jnp.where(qcg > 0, sidx % jnp.maximum(qcg, 1), 0)
    rows = jnp.take_along_axis(rows, src, axis=1)
    return ge, rows, qc


def kernel(x, expert_index, down_w, down_b, up_w):
    B, S, C = x.shape
    M, N, _, D = down_w.shape
    NG = (B + (_G - 1) * N) // _G

    ge, rows, qc = _routing(expert_index, N, NG)
    db4 = down_b.reshape(M, N, 1, D)

    grid_spec = pltpu.PrefetchScalarGridSpec(
        num_scalar_prefetch=3,
        grid=(M, NG),
        in_specs=[
            pl.BlockSpec((B, S, C), lambda m, g, ge, rw, qc: (0, 0, 0)),
            pl.BlockSpec((1, 1, C, D), lambda m, g, ge, rw, qc: (m, ge[m, g], 0, 0)),
            pl.BlockSpec((1, 1, 1, D), lambda m, g, ge, rw, qc: (m, ge[m, g], 0, 0)),
            pl.BlockSpec((1, 1, D, C), lambda m, g, ge, rw, qc: (m, ge[m, g], 0, 0)),
        ],
        out_specs=pl.BlockSpec((1, B, S, C), lambda m, g, ge, rw, qc: (m, 0, 0, 0)),
    )

    out = pl.pallas_call(
        _body,
        grid_spec=grid_spec,
        out_shape=jax.ShapeDtypeStruct((M, B, S, C), jnp.float32),
        compiler_params=pltpu.CompilerParams(
            dimension_semantics=("arbitrary", "arbitrary"),
        ),
    )(ge, rows, qc, x, down_w, db4, up_w)
    return out


# speedup vs baseline: 1.5849x; 1.1603x over previous
import jax
import jax.numpy as jnp
from jax.experimental import pallas as pl
from jax.experimental.pallas import tpu as pltpu

_G = 8


def _body(ge_ref, rows_ref, qc_ref, x_ref, dw_ref, db_ref, uw_ref, o_ref):
    m = pl.program_id(0)
    g = pl.program_id(1)
    S = x_ref.shape[1]

    @pl.when(qc_ref[m, g] > 0)
    def _():
        rows = [rows_ref[m, g * _G + i] for i in range(_G)]
        xt = jnp.concatenate([x_ref[r] for r in rows], axis=0)
        z = jnp.dot(xt, dw_ref[0, 0], preferred_element_type=jnp.float32)
        z = z + db_ref[0, 0, 0][None, :]
        z = z * jax.nn.sigmoid(z)
        u = jnp.dot(z, uw_ref[0, 0], preferred_element_type=jnp.float32)
        for i in range(_G):
            o_ref[0, rows[i]] = u[i * S:(i + 1) * S]


def _routing(expert_index, N, NG):
    M, B = expert_index.shape
    iN = jnp.arange(N, dtype=jnp.int32)
    oh = jax.nn.one_hot(expert_index, N, dtype=jnp.int32)
    counts = jnp.sum(oh, axis=1)
    gsz = (counts + _G - 1) // _G
    estart = jnp.cumsum(counts, axis=1) - counts
    gbefore = jnp.cumsum(gsz, axis=1) - gsz
    ngroups = jnp.sum(gsz, axis=1)

    ohcum = jnp.cumsum(oh, axis=1)
    within = jnp.einsum('mbn,mbn->mb', ohcum, oh) - 1
    pos = jnp.einsum('mbn,mn->mb', oh, estart) + within
    pos_oh = jax.nn.one_hot(pos, B, dtype=jnp.int32)
    order = jnp.einsum('mbp,b->mp', pos_oh, jnp.arange(B, dtype=jnp.int32))

    g = jnp.arange(NG)[None, None, :]
    in_e = ((g >= gbefore[:, :, None])
            & (g < (gbefore + gsz)[:, :, None])).astype(jnp.int32)
    ge = jnp.einsum('mng,n->mg', in_e, iN)
    last_e = jnp.max(jnp.where(counts > 0, iN[None, :], -1), axis=1)
    gidx = jnp.arange(NG, dtype=jnp.int32)[None, :]
    ge = jnp.where(gidx < ngroups[:, None], ge, last_e[:, None]).astype(jnp.int32)

    cnt_g = jnp.einsum('mng,mn->mg', in_e, counts)
    gb_g = jnp.einsum('mng,mn->mg', in_e, gbefore)
    es_g = jnp.einsum('mng,mn->mg', in_e, estart)
    qc = jnp.clip(cnt_g - (gidx - gb_g) * _G, 0, _G).astype(jnp.int32)

    sI = jnp.arange(NG * _G, dtype=jnp.int32)[None, :] % _G
    qc_r = jnp.repeat(qc, _G, axis=1)
    posg = (jnp.repeat(es_g, _G, axis=1)
            + (jnp.arange(NG * _G)[None, :] // _G - jnp.repeat(gb_g, _G, axis=1)) * _G
            + jnp.where(qc_r > 0, sI % jnp.maximum(qc_r, 1), 0))
    posg = jnp.clip(posg, 0, B - 1)
    rows = jnp.einsum('mib,mb->mi', jax.nn.one_hot(posg, B, dtype=jnp.int32),
                      order).astype(jnp.int32)
    return ge, rows, qc


def kernel(x, expert_index, down_w, down_b, up_w):
    B, S, C = x.shape
    M, N, _, D = down_w.shape
    NG = (B + (_G - 1) * N) // _G

    ge, rows, qc = _routing(expert_index, N, NG)
    db4 = down_b.reshape(M, N, 1, D)

    grid_spec = pltpu.PrefetchScalarGridSpec(
        num_scalar_prefetch=3,
        grid=(M, NG),
        in_specs=[
            pl.BlockSpec((B, S, C), lambda m, g, ge, rw, qc: (0, 0, 0)),
            pl.BlockSpec((1, 1, C, D), lambda m, g, ge, rw, qc: (m, ge[m, g], 0, 0)),
            pl.BlockSpec((1, 1, 1, D), lambda m, g, ge, rw, qc: (m, ge[m, g], 0, 0)),
            pl.BlockSpec((1, 1, D, C), lambda m, g, ge, rw, qc: (m, ge[m, g], 0, 0)),
        ],
        out_specs=pl.BlockSpec((1, B, S, C), lambda m, g, ge, rw, qc: (m, 0, 0, 0)),
    )

    out = pl.pallas_call(
        _body,
        grid_spec=grid_spec,
        out_shape=jax.ShapeDtypeStruct((M, B, S, C), jnp.float32),
        compiler_params=pltpu.CompilerParams(
            dimension_semantics=("arbitrary", "arbitrary"),
        ),
    )(ge, rows, qc, x, down_w, db4, up_w)
    return out


# speedup vs baseline: 2.3614x; 1.4899x over previous
import jax
import jax.numpy as jnp
from jax.experimental import pallas as pl
from jax.experimental.pallas import tpu as pltpu

_G = 8
_K = 8


def _body(tp_ref, ge_ref, rows_ref, x_ref, dw_ref, db_ref, uw_ref, o_ref):
    m = pl.program_id(0)
    ck = pl.program_id(1)
    S = x_ref.shape[1]

    def tile(t, carry):
        e_local = ge_ref[m, t] - ck * _K
        rows = [rows_ref[m, t * _G + i] for i in range(_G)]
        xt = jnp.concatenate([x_ref[r] for r in rows], axis=0)
        z = jnp.dot(xt, dw_ref[0, e_local],
                    preferred_element_type=jnp.float32)
        z = z + db_ref[0, e_local, 0][None, :]
        z = z * jax.nn.sigmoid(z)
        u = jnp.dot(z, uw_ref[0, e_local],
                    preferred_element_type=jnp.float32)
        for i in range(_G):
            o_ref[0, rows[i]] = u[i * S:(i + 1) * S]
        return carry

    jax.lax.fori_loop(tp_ref[m, ck], tp_ref[m, ck + 1], tile, 0)


def _routing(expert_index, N, NG):
    M, B = expert_index.shape
    iN = jnp.arange(N, dtype=jnp.int32)
    oh = jax.nn.one_hot(expert_index, N, dtype=jnp.int32)
    counts = jnp.sum(oh, axis=1)
    gsz = (counts + _G - 1) // _G
    estart = jnp.cumsum(counts, axis=1) - counts
    gbefore = jnp.cumsum(gsz, axis=1) - gsz
    ngroups = jnp.sum(gsz, axis=1)

    ohcum = jnp.cumsum(oh, axis=1)
    within = jnp.einsum('mbn,mbn->mb', ohcum, oh) - 1
    pos = jnp.einsum('mbn,mn->mb', oh, estart) + within
    pos_oh = jax.nn.one_hot(pos, B, dtype=jnp.int32)
    order = jnp.einsum('mbp,b->mp', pos_oh, jnp.arange(B, dtype=jnp.int32))

    g = jnp.arange(NG)[None, None, :]
    in_e = ((g >= gbefore[:, :, None])
            & (g < (gbefore + gsz)[:, :, None])).astype(jnp.int32)
    ge = jnp.einsum('mng,n->mg', in_e, iN).astype(jnp.int32)
    gidx = jnp.arange(NG, dtype=jnp.int32)[None, :]

    cnt_g = jnp.einsum('mng,mn->mg', in_e, counts)
    gb_g = jnp.einsum('mng,mn->mg', in_e, gbefore)
    es_g = jnp.einsum('mng,mn->mg', in_e, estart)
    qc = jnp.clip(cnt_g - (gidx - gb_g) * _G, 0, _G)

    sI = jnp.arange(NG * _G, dtype=jnp.int32)[None, :] % _G
    qc_r = jnp.repeat(qc, _G, axis=1)
    posg = (jnp.repeat(es_g, _G, axis=1)
            + (jnp.arange(NG * _G)[None, :] // _G - jnp.repeat(gb_g, _G, axis=1)) * _G
            + jnp.where(qc_r > 0, sI % jnp.maximum(qc_r, 1), 0))
    posg = jnp.clip(posg, 0, B - 1)
    rows = jnp.einsum('mib,mb->mi', jax.nn.one_hot(posg, B, dtype=jnp.int32),
                      order).astype(jnp.int32)

    gb_ext = jnp.concatenate([gbefore, ngroups[:, None]], axis=1)
    tp = gb_ext[:, ::_K].astype(jnp.int32)
    return tp, ge, rows


def kernel(x, expert_index, down_w, down_b, up_w):
    B, S, C = x.shape
    M, N, _, D = down_w.shape
    NG = (B + (_G - 1) * N) // _G
    NC = N // _K

    tp, ge, rows = _routing(expert_index, N, NG)
    db4 = down_b.reshape(M, N, 1, D)

    grid_spec = pltpu.PrefetchScalarGridSpec(
        num_scalar_prefetch=3,
        grid=(M, NC),
        in_specs=[
            pl.BlockSpec((B, S, C), lambda m, ck, tp, ge, rw: (0, 0, 0)),
            pl.BlockSpec((1, _K, C, D), lambda m, ck, tp, ge, rw: (m, ck, 0, 0)),
            pl.BlockSpec((1, _K, 1, D), lambda m, ck, tp, ge, rw: (m, ck, 0, 0)),
            pl.BlockSpec((1, _K, D, C), lambda m, ck, tp, ge, rw: (m, ck, 0, 0)),
        ],
        out_specs=pl.BlockSpec((1, B, S, C), lambda m, ck, tp, ge, rw: (m, 0, 0, 0)),
    )

    out = pl.pallas_call(
        _body,
        grid_spec=grid_spec,
        out_shape=jax.ShapeDtypeStruct((M, B, S, C), jnp.float32),
        compiler_params=pltpu.CompilerParams(
            dimension_semantics=("arbitrary", "arbitrary"),
        ),
    )(tp, ge, rows, x, down_w, db4, up_w)
    return out


# speedup vs baseline: 2.5154x; 1.0652x over previous
import jax
import jax.numpy as jnp
from jax.experimental import pallas as pl
from jax.experimental.pallas import tpu as pltpu

_G = 8
_K = 8


def _body(tp_ref, ge_ref, rows_ref, x_ref, dw_ref, db_ref, uw_ref, o_ref):
    m = pl.program_id(0)
    ck = pl.program_id(1)
    S = x_ref.shape[1]

    t0 = tp_ref[m, ck]
    t1 = tp_ref[m, ck + 1]

    def down(t):
        e_local = ge_ref[m, t] - ck * _K
        rows = tuple(rows_ref[m, t * _G + i] for i in range(_G))
        xt = jnp.concatenate([x_ref[r] for r in rows], axis=0)
        z = jnp.dot(xt, dw_ref[0, e_local],
                    preferred_element_type=jnp.float32)
        z = z + db_ref[0, e_local, 0][None, :]
        return z * jax.nn.sigmoid(z), e_local, rows

    def up_store(z, e_local, rows):
        u = jnp.dot(z, uw_ref[0, e_local],
                    preferred_element_type=jnp.float32)
        for i in range(_G):
            o_ref[0, rows[i]] = u[i * S:(i + 1) * S]

    @pl.when(t1 > t0)
    def _():
        def step(t, carry):
            nxt = down(t)
            up_store(*carry)
            return nxt

        last = jax.lax.fori_loop(t0 + 1, t1, step, down(t0))
        up_store(*last)


def _routing(expert_index, N, NG):
    M, B = expert_index.shape
    iN = jnp.arange(N, dtype=jnp.int32)
    oh = jax.nn.one_hot(expert_index, N, dtype=jnp.int32)
    counts = jnp.sum(oh, axis=1)
    gsz = (counts + _G - 1) // _G
    estart = jnp.cumsum(counts, axis=1) - counts
    gbefore = jnp.cumsum(gsz, axis=1) - gsz
    ngroups = jnp.sum(gsz, axis=1)

    ohcum = jnp.cumsum(oh, axis=1)
    within = jnp.einsum('mbn,mbn->mb', ohcum, oh) - 1
    pos = jnp.einsum('mbn,mn->mb', oh, estart) + within
    pos_oh = jax.nn.one_hot(pos, B, dtype=jnp.int32)
    order = jnp.einsum('mbp,b->mp', pos_oh, jnp.arange(B, dtype=jnp.int32))

    g = jnp.arange(NG)[None, None, :]
    in_e = ((g >= gbefore[:, :, None])
            & (g < (gbefore + gsz)[:, :, None])).astype(jnp.int32)
    ge = jnp.einsum('mng,n->mg', in_e, iN).astype(jnp.int32)
    gidx = jnp.arange(NG, dtype=jnp.int32)[None, :]

    cnt_g = jnp.einsum('mng,mn->mg', in_e, counts)
    gb_g = jnp.einsum('mng,mn->mg', in_e, gbefore)
    es_g = jnp.einsum('mng,mn->mg', in_e, estart)
    qc = jnp.clip(cnt_g - (gidx - gb_g) * _G, 0, _G)

    sI = jnp.arange(NG * _G, dtype=jnp.int32)[None, :] % _G
    qc_r = jnp.repeat(qc, _G, axis=1)
    posg = (jnp.repeat(es_g, _G, axis=1)
            + (jnp.arange(NG * _G)[None, :] // _G - jnp.repeat(gb_g, _G, axis=1)) * _G
            + jnp.where(qc_r > 0, sI % jnp.maximum(qc_r, 1), 0))
    posg = jnp.clip(posg, 0, B - 1)
    rows = jnp.einsum('mib,mb->mi', jax.nn.one_hot(posg, B, dtype=jnp.int32),
                      order).astype(jnp.int32)

    gb_ext = jnp.concatenate([gbefore, ngroups[:, None]], axis=1)
    tp = gb_ext[:, ::_K].astype(jnp.int32)
    return tp, ge, rows


def kernel(x, expert_index, down_w, down_b, up_w):
    B, S, C = x.shape
    M, N, _, D = down_w.shape
    NG = (B + (_G - 1) * N) // _G
    NC = N // _K

    tp, ge, rows = _routing(expert_index, N, NG)
    db4 = down_b.reshape(M, N, 1, D)

    grid_spec = pltpu.PrefetchScalarGridSpec(
        num_scalar_prefetch=3,
        grid=(M, NC),
        in_specs=[
            pl.BlockSpec((B, S, C), lambda m, ck, tp, ge, rw: (0, 0, 0)),
            pl.BlockSpec((1, _K, C, D), lambda m, ck, tp, ge, rw: (m, ck, 0, 0)),
            pl.BlockSpec((1, _K, 1, D), lambda m, ck, tp, ge, rw: (m, ck, 0, 0)),
            pl.BlockSpec((1, _K, D, C), lambda m, ck, tp, ge, rw: (m, ck, 0, 0)),
        ],
        out_specs=pl.BlockSpec((1, B, S, C), lambda m, ck, tp, ge, rw: (m, 0, 0, 0)),
    )

    out = pl.pallas_call(
        _body,
        grid_spec=grid_spec,
        out_shape=jax.ShapeDtypeStruct((M, B, S, C), jnp.float32),
        compiler_params=pltpu.CompilerParams(
            dimension_semantics=("arbitrary", "arbitrary"),
        ),
    )(tp, ge, rows, x, down_w, db4, up_w)
    return out


# speedup vs baseline: 2.7368x; 1.0880x over previous
import jax
import jax.numpy as jnp
from jax.experimental import pallas as pl
from jax.experimental.pallas import tpu as pltpu

_G = 8
_K = 8
_NBUF = 4


def _body(tp_ref, ge_ref, rows_ref, x_ref, dw_hbm, db_ref, uw_hbm, o_ref,
          dwb_ref, uwb_ref, sem):
    m = pl.program_id(0)
    ck = pl.program_id(1)
    NC = pl.num_programs(1)
    S = x_ref.shape[1]
    g = m * NC + ck
    total = pl.num_programs(0) * NC

    def copies(gg, slot):
        mm = gg // NC
        cc = gg % NC
        return (pltpu.make_async_copy(dw_hbm.at[mm, pl.ds(cc * _K, _K)],
                                      dwb_ref.at[slot], sem.at[slot, 0]),
                pltpu.make_async_copy(uw_hbm.at[mm, pl.ds(cc * _K, _K)],
                                      uwb_ref.at[slot], sem.at[slot, 1]))

    @pl.when(g == 0)
    def _():
        for l in range(_NBUF - 1):
            for c in copies(l, l):
                c.start()

    @pl.when(g + _NBUF - 1 < total)
    def _():
        gg = g + _NBUF - 1
        for c in copies(gg, gg % _NBUF):
            c.start()

    slot = g % _NBUF
    for c in copies(g, slot):
        c.wait()

    t0 = tp_ref[m, ck]
    t1 = tp_ref[m, ck + 1]

    def down(t):
        e_local = ge_ref[m, t] - ck * _K
        rows = tuple(rows_ref[m, t * _G + i] for i in range(_G))
        xt = jnp.concatenate([x_ref[r] for r in rows], axis=0)
        z = jnp.dot(xt, dwb_ref[slot, e_local],
                    preferred_element_type=jnp.float32)
        z = z + db_ref[0, e_local, 0][None, :]
        return z * jax.nn.sigmoid(z), e_local, rows

    def up_store(z, e_local, rows):
        u = jnp.dot(z, uwb_ref[slot, e_local],
                    preferred_element_type=jnp.float32)
        for i in range(_G):
            o_ref[0, rows[i]] = u[i * S:(i + 1) * S]

    @pl.when(t1 > t0)
    def _():
        def step(t, carry):
            nxt = down(t)
            up_store(*carry)
            return nxt

        last = jax.lax.fori_loop(t0 + 1, t1, step, down(t0))
        up_store(*last)


def _routing(expert_index, N, NG):
    M, B = expert_index.shape
    iN = jnp.arange(N, dtype=jnp.int32)
    oh = jax.nn.one_hot(expert_index, N, dtype=jnp.int32)
    counts = jnp.sum(oh, axis=1)
    gsz = (counts + _G - 1) // _G
    estart = jnp.cumsum(counts, axis=1) - counts
    gbefore = jnp.cumsum(gsz, axis=1) - gsz
    ngroups = jnp.sum(gsz, axis=1)

    ohcum = jnp.cumsum(oh, axis=1)
    within = jnp.einsum('mbn,mbn->mb', ohcum, oh) - 1
    pos = jnp.einsum('mbn,mn->mb', oh, estart) + within
    pos_oh = jax.nn.one_hot(pos, B, dtype=jnp.int32)
    order = jnp.einsum('mbp,b->mp', pos_oh, jnp.arange(B, dtype=jnp.int32))

    g = jnp.arange(NG)[None, None, :]
    in_e = ((g >= gbefore[:, :, None])
            & (g < (gbefore + gsz)[:, :, None])).astype(jnp.int32)
    ge = jnp.einsum('mng,n->mg', in_e, iN).astype(jnp.int32)
    gidx = jnp.arange(NG, dtype=jnp.int32)[None, :]

    cnt_g = jnp.einsum('mng,mn->mg', in_e, counts)
    gb_g = jnp.einsum('mng,mn->mg', in_e, gbefore)
    es_g = jnp.einsum('mng,mn->mg', in_e, estart)
    qc = jnp.clip(cnt_g - (gidx - gb_g) * _G, 0, _G)

    sI = jnp.arange(NG * _G, dtype=jnp.int32)[None, :] % _G
    qc_r = jnp.repeat(qc, _G, axis=1)
    posg = (jnp.repeat(es_g, _G, axis=1)
            + (jnp.arange(NG * _G)[None, :] // _G - jnp.repeat(gb_g, _G, axis=1)) * _G
            + jnp.where(qc_r > 0, sI % jnp.maximum(qc_r, 1), 0))
    posg = jnp.clip(posg, 0, B - 1)
    rows = jnp.einsum('mib,mb->mi', jax.nn.one_hot(posg, B, dtype=jnp.int32),
                      order).astype(jnp.int32)

    gb_ext = jnp.concatenate([gbefore, ngroups[:, None]], axis=1)
    tp = gb_ext[:, ::_K].astype(jnp.int32)
    return tp, ge, rows


def kernel(x, expert_index, down_w, down_b, up_w):
    B, S, C = x.shape
    M, N, _, D = down_w.shape
    NG = (B + (_G - 1) * N) // _G
    NC = N // _K

    tp, ge, rows = _routing(expert_index, N, NG)
    db4 = down_b.reshape(M, N, 1, D)

    grid_spec = pltpu.PrefetchScalarGridSpec(
        num_scalar_prefetch=3,
        grid=(M, NC),
        in_specs=[
            pl.BlockSpec((B, S, C), lambda m, ck, tp, ge, rw: (0, 0, 0)),
            pl.BlockSpec(memory_space=pl.ANY),
            pl.BlockSpec((1, _K, 1, D), lambda m, ck, tp, ge, rw: (m, ck, 0, 0)),
            pl.BlockSpec(memory_space=pl.ANY),
        ],
        out_specs=pl.BlockSpec((1, B, S, C), lambda m, ck, tp, ge, rw: (m, 0, 0, 0)),
        scratch_shapes=[
            pltpu.VMEM((_NBUF, _K, C, D), jnp.float32),
            pltpu.VMEM((_NBUF, _K, D, C), jnp.float32),
            pltpu.SemaphoreType.DMA((_NBUF, 2)),
        ],
    )

    out = pl.pallas_call(
        _body,
        grid_spec=grid_spec,
        out_shape=jax.ShapeDtypeStruct((M, B, S, C), jnp.float32),
        compiler_params=pltpu.CompilerParams(
            dimension_semantics=("arbitrary", "arbitrary"),
        ),
    )(tp, ge, rows, x, down_w, db4, up_w)
    return out
